# baseline (speedup 1.0000x reference)
import functools

import jax
import jax.numpy as jnp
from jax import lax
from jax.experimental import pallas as pl
from jax.experimental.pallas import tpu as pltpu
from jax.experimental.pallas import tpu_sc as plsc

EPS = 1e-5


def _make_gather_kernel(B, D, num_cores, num_subcores):
    NW = num_cores * num_subcores
    b_per_w = B // NW
    CH = 128
    n_ch = b_per_w // CH
    D2 = 2 * D
    mesh = plsc.VectorSubcoreMesh(core_axis_name="c", subcore_axis_name="s")

    @functools.partial(
        pl.kernel,
        mesh=mesh,
        out_type=(
            jax.ShapeDtypeStruct((B, D), jnp.float32),
            jax.ShapeDtypeStruct((B, D), jnp.float32),
        ),
        scratch_types=[
            pltpu.VMEM((n_ch, CH), jnp.int32),
            pltpu.VMEM((n_ch, CH), jnp.int32),
            pltpu.VMEM((n_ch, CH), jnp.int32),
            pltpu.VMEM((n_ch, CH), jnp.int32),
            pltpu.VMEM((2, CH, D2), jnp.float32),
            pltpu.VMEM((2, CH, D2), jnp.float32),
            pltpu.VMEM((CH, D), jnp.float32),
            pltpu.VMEM((CH, D), jnp.float32),
            pltpu.SemaphoreType.DMA,
            pltpu.SemaphoreType.DMA,
        ],
    )
    def gather_k(uid_hbm, mid_hbm, utab_hbm, mtab_hbm, uout_hbm, mout_hbm,
                 uidx_v, midx_v, usft_v, msft_v, ubuf_v, mbuf_v,
                 urow_v, mrow_v, usem, msem):
        wid = lax.axis_index("s") * num_cores + lax.axis_index("c")
        base = wid * b_per_w
        row0 = wid * n_ch
        pltpu.sync_copy(uid_hbm.at[pl.ds(row0, n_ch)], uidx_v)
        pltpu.sync_copy(mid_hbm.at[pl.ds(row0, n_ch)], midx_v)
        for j in range(n_ch):
            for k in range(CH // 16):
                sl = pl.ds(k * 16, 16)
                usft_v[j, sl] = lax.shift_right_logical(uidx_v[j, sl], 1)
                msft_v[j, sl] = lax.shift_right_logical(midx_v[j, sl], 1)

        def issue(j, buf_slot):
            cu = pltpu.async_copy(
                utab_hbm.at[usft_v.at[j]], ubuf_v.at[buf_slot], usem)
            cm = pltpu.async_copy(
                mtab_hbm.at[msft_v.at[j]], mbuf_v.at[buf_slot], msem)
            return cu, cm

        def extract(idx_v, j, buf, row_ref):
            def body(g):
                par = (idx_v[j, pl.ds(g * 16, 16)] & 1) * D
                for l in range(16):
                    r = g * 16 + l
                    off = par[l]
                    for c in range(D // 16):
                        row_ref[r, pl.ds(c * 16, 16)] = (
                            buf[r, pl.ds(off + c * 16, 16)])
            pl.loop(0, CH // 16)(body)

        pend = issue(0, 0)
        for j in range(n_ch):
            nxt = issue(j + 1, (j + 1) % 2) if j + 1 < n_ch else None
            cu, cm = pend
            out_sl = pl.ds(base + j * CH, CH)
            cu.wait()
            extract(uidx_v, j, ubuf_v.at[j % 2], urow_v)
            pltpu.sync_copy(urow_v, uout_hbm.at[out_sl])
            cm.wait()
            extract(midx_v, j, mbuf_v.at[j % 2], mrow_v)
            pltpu.sync_copy(mrow_v, mout_hbm.at[out_sl])
            pend = nxt

    return gather_k


def _mlp_body(u_ref, m_ref, W1_ref, b1_ref, g1_ref, bt1_ref,
              W2_ref, b2_ref, g2_ref, bt2_ref, W3_ref, b3_ref, o_ref):
    c = 1.0 / (1.0 + EPS) ** 0.5
    u = u_ref[...]
    m = m_ref[...]
    D = u.shape[1]
    h = jnp.dot(u, W1_ref[:D, :], preferred_element_type=jnp.float32)
    h += jnp.dot(m, W1_ref[D:, :], preferred_element_type=jnp.float32)
    h = (h + b1_ref[...]) * (g1_ref[...] * c) + bt1_ref[...]
    h = jnp.maximum(h, 0.0)
    h = jnp.dot(h, W2_ref[...], preferred_element_type=jnp.float32)
    h = (h + b2_ref[...]) * (g2_ref[...] * c) + bt2_ref[...]
    h = jnp.maximum(h, 0.0)
    o = jnp.dot(h, W3_ref[...], preferred_element_type=jnp.float32)
    o_ref[...] = o + b3_ref[...]


def kernel(user_ids, movie_ids, user_table, movie_table,
           W1, b1, g1, beta1, W2, b2, g2, beta2, W3, b3):
    B = user_ids.shape[0]
    D = user_table.shape[1]
    H1 = W1.shape[1]
    H2 = W2.shape[1]

    info = plsc.get_sparse_core_info()
    gather_k = _make_gather_kernel(B, D, info.num_cores, info.num_subcores)
    uid2d = user_ids.reshape(-1, 128)
    mid2d = movie_ids.reshape(-1, 128)
    utab2 = user_table.reshape(-1, 2 * D)
    mtab2 = movie_table.reshape(-1, 2 * D)
    u_emb, m_emb = gather_k(uid2d, mid2d, utab2, mtab2)

    BLK = 2048
    nblk = B // BLK
    row2d = lambda v: v.reshape(1, -1)
    full = lambda shape: pl.BlockSpec(shape, lambda i: (0, 0))

    out = pl.pallas_call(
        _mlp_body,
        grid=(nblk,),
        in_specs=[
            pl.BlockSpec((BLK, D), lambda i: (i, 0)),
            pl.BlockSpec((BLK, D), lambda i: (i, 0)),
            full((2 * D, H1)),
            full((1, H1)), full((1, H1)), full((1, H1)),
            full((H1, H2)),
            full((1, H2)), full((1, H2)), full((1, H2)),
            full((H2, 1)),
            full((1, 1)),
        ],
        out_specs=pl.BlockSpec((BLK, 1), lambda i: (i, 0)),
        out_shape=jax.ShapeDtypeStruct((B, 1), jnp.float32),
    )(u_emb, m_emb, W1, row2d(b1), row2d(g1), row2d(beta1),
      W2, row2d(b2), row2d(g2), row2d(beta2), W3, row2d(b3))
    return out[:, 0]


# speedup vs baseline: 1.6506x; 1.6506x over previous
import functools

import jax
import jax.numpy as jnp
from jax import lax
from jax.experimental import pallas as pl
from jax.experimental.pallas import tpu as pltpu
from jax.experimental.pallas import tpu_sc as plsc

EPS = 1e-5


def _make_gather_kernel(B, D, num_cores, num_subcores):
    NW = num_cores * num_subcores
    b_per_w = B // NW
    G = 16
    HALVES = 2
    b_half = b_per_w // HALVES
    n_gh = b_half // G
    LOOKAHEAD = 4
    mesh = plsc.VectorSubcoreMesh(core_axis_name="c", subcore_axis_name="s")

    @functools.partial(
        pl.kernel,
        mesh=mesh,
        out_type=(
            jax.ShapeDtypeStruct((B, D), jnp.float32),
            jax.ShapeDtypeStruct((B, D), jnp.float32),
        ),
        scratch_types=[
            pltpu.VMEM((b_per_w,), jnp.int32),
            pltpu.VMEM((b_per_w,), jnp.int32),
            pltpu.VMEM((b_half, D), jnp.float32),
            pltpu.VMEM((b_half, D), jnp.float32),
            pltpu.SemaphoreType.DMA,
            pltpu.SemaphoreType.DMA,
        ],
    )
    def gather_k(uid_hbm, mid_hbm, utab_hbm, mtab_hbm, uout_hbm, mout_hbm,
                 uidx_v, midx_v, urows_v, mrows_v, usem, msem):
        wid = lax.axis_index("s") * num_cores + lax.axis_index("c")
        base = wid * b_per_w
        pltpu.sync_copy(uid_hbm.at[pl.ds(base, b_per_w)], uidx_v)
        pltpu.sync_copy(mid_hbm.at[pl.ds(base, b_per_w)], midx_v)

        def issue_group(h, g):
            u16 = uidx_v[pl.ds((h * n_gh + g) * G, G)]
            m16 = midx_v[pl.ds((h * n_gh + g) * G, G)]
            for l in range(G):
                r = g * G + l
                pltpu.async_copy(
                    utab_hbm.at[pl.ds(u16[l], 1)], urows_v.at[pl.ds(r, 1)], usem)
                pltpu.async_copy(
                    mtab_hbm.at[pl.ds(m16[l], 1)], mrows_v.at[pl.ds(r, 1)], msem)

        def drain(sem, rows_ref, tab_ref, n_rows):
            pltpu.make_async_copy(
                tab_ref.at[pl.ds(0, n_rows)],
                rows_ref.at[pl.ds(0, n_rows)], sem).wait()

        for h in range(HALVES):
            for g in range(LOOKAHEAD):
                issue_group(h, g)

            def body(g, _h=h):
                drain(usem, urows_v, utab_hbm, G)
                drain(msem, mrows_v, mtab_hbm, G)
                issue_group(_h, g)

            pl.loop(LOOKAHEAD, n_gh)(body)
            drain(usem, urows_v, utab_hbm, LOOKAHEAD * G)
            drain(msem, mrows_v, mtab_hbm, LOOKAHEAD * G)
            out_sl = pl.ds(base + h * b_half, b_half)
            pltpu.sync_copy(urows_v, uout_hbm.at[out_sl])
            pltpu.sync_copy(mrows_v, mout_hbm.at[out_sl])

    return gather_k


def _mlp_body(u_ref, m_ref, W1_ref, b1_ref, g1_ref, bt1_ref,
              W2_ref, b2_ref, g2_ref, bt2_ref, W3_ref, b3_ref, o_ref):
    c = 1.0 / (1.0 + EPS) ** 0.5
    u = u_ref[...]
    m = m_ref[...]
    D = u.shape[1]
    h = jnp.dot(u, W1_ref[:D, :], preferred_element_type=jnp.float32)
    h += jnp.dot(m, W1_ref[D:, :], preferred_element_type=jnp.float32)
    h = (h + b1_ref[...]) * (g1_ref[...] * c) + bt1_ref[...]
    h = jnp.maximum(h, 0.0)
    h = jnp.dot(h, W2_ref[...], preferred_element_type=jnp.float32)
    h = (h + b2_ref[...]) * (g2_ref[...] * c) + bt2_ref[...]
    h = jnp.maximum(h, 0.0)
    o = jnp.dot(h, W3_ref[...], preferred_element_type=jnp.float32)
    o_ref[...] = o + b3_ref[...]


def kernel(user_ids, movie_ids, user_table, movie_table,
           W1, b1, g1, beta1, W2, b2, g2, beta2, W3, b3):
    B = user_ids.shape[0]
    D = user_table.shape[1]
    H1 = W1.shape[1]
    H2 = W2.shape[1]

    info = plsc.get_sparse_core_info()
    gather_k = _make_gather_kernel(B, D, info.num_cores, info.num_subcores)
    u_emb, m_emb = gather_k(user_ids, movie_ids, user_table, movie_table)

    BLK = 2048
    nblk = B // BLK
    row2d = lambda v: v.reshape(1, -1)
    full = lambda shape: pl.BlockSpec(shape, lambda i: (0, 0))

    out = pl.pallas_call(
        _mlp_body,
        grid=(nblk,),
        in_specs=[
            pl.BlockSpec((BLK, D), lambda i: (i, 0)),
            pl.BlockSpec((BLK, D), lambda i: (i, 0)),
            full((2 * D, H1)),
            full((1, H1)), full((1, H1)), full((1, H1)),
            full((H1, H2)),
            full((1, H2)), full((1, H2)), full((1, H2)),
            full((H2, 1)),
            full((1, 1)),
        ],
        out_specs=pl.BlockSpec((BLK, 1), lambda i: (i, 0)),
        out_shape=jax.ShapeDtypeStruct((B, 1), jnp.float32),
    )(u_emb, m_emb, W1, row2d(b1), row2d(g1), row2d(beta1),
      W2, row2d(b2), row2d(g2), row2d(beta2), W3, row2d(b3))
    return out[:, 0]
